# baseline (speedup 1.0000x reference)
import functools
import math

import jax
import jax.numpy as jnp
from jax import lax
from jax.experimental import pallas as pl
from jax.experimental.pallas import tpu as pltpu
from jax.experimental.pallas import tpu_sc as plsc

VOCAB = 1000000
EMB = 64
SCALE = math.sqrt(EMB)

NC = 2
NS = 16
NW = NC * NS

S = 4096
T = 200

NPB = VOCAB // 256 * 128 + 64
NBF = VOCAB // 128
TAIL_COLS = VOCAB - NBF * 128
NG1 = 123


def _iota16():
    return lax.iota(jnp.int32, 16)


def _pack_kernel(tabT, tail2, packed, inb0, inb1, outb0, outb1,
                 isem0, isem1, osem0, osem1):
    w = lax.axis_index("s") * NC + lax.axis_index("c")
    inbs = (inb0, inb1)
    outbs = (outb0, outb1)
    isems = (isem0, isem1)
    osems = (osem0, osem1)

    def blk(k):
        return w + 32 * k

    def valid(k):
        return blk(k) <= NBF - 1

    r_idx4 = [_iota16() + 16 * k for k in range(4)]

    def transpose_pack(inb, outb, nrows):
        def ibody(i, c):
            c_even = jnp.broadcast_to(2 * i, (16,)).astype(jnp.int32)
            c_odd = c_even + 1
            for k in range(8):
                r_idx = r_idx4[k % 4]
                c_idx = c_even if k < 4 else c_odd
                v = plsc.load_gather(inb, [r_idx, c_idx])
                outb[i, pl.ds(16 * k, 16)] = v
            return c
        lax.fori_loop(0, nrows, ibody, 0, unroll=4)

    def inb_dst(inb):
        return inb.at[:, pl.ds(0, 128)]

    @pl.when(valid(0))
    def _():
        pltpu.async_copy(
            tabT.at[:, pl.ds(blk(0) * 128, 128)], inb_dst(inb0), isem0)

    def group(g, carry):
        for b in range(2):
            k = g * 2 + b
            inb, outb, isem, osem = inbs[b], outbs[b], isems[b], osems[b]

            @pl.when(valid(k))
            def _():
                v0 = blk(k)
                pltpu.make_async_copy(
                    tabT.at[:, pl.ds(v0 * 128, 128)], inb_dst(inb),
                    isem).wait()

                @pl.when(valid(k + 1))
                def _():
                    pltpu.async_copy(
                        tabT.at[:, pl.ds(blk(k + 1) * 128, 128)],
                        inb_dst(inbs[1 - b]), isems[1 - b])

                @pl.when(k >= 2)
                def _():
                    pltpu.make_async_copy(
                        outb, packed.at[pl.ds(blk(k - 2) * 64, 64)], osem
                    ).wait()

                transpose_pack(inb, outb, 64)
                pltpu.async_copy(outb, packed.at[pl.ds(v0 * 64, 64)], osem)
        return carry

    lax.fori_loop(0, NG1, group, 0)

    @pl.when(w < 4)
    def _():
        pltpu.make_async_copy(
            outb1, packed.at[pl.ds(blk(243) * 64, 64)], osem1).wait()
        pltpu.make_async_copy(
            outb0, packed.at[pl.ds(blk(244) * 64, 64)], osem0).wait()

    @pl.when(w >= 4)
    def _():
        pltpu.make_async_copy(
            outb0, packed.at[pl.ds(blk(242) * 64, 64)], osem0).wait()
        pltpu.make_async_copy(
            outb1, packed.at[pl.ds(blk(243) * 64, 64)], osem1).wait()

    @pl.when(w == 4)
    def _():
        pltpu.async_copy(
            tail2, inb0.at[pl.ds(0, TAIL_COLS // 2), pl.ds(0, 128)],
            isem0).wait()
        pltpu.async_copy(
            inb0.at[pl.ds(0, TAIL_COLS // 2), pl.ds(0, 128)],
            packed.at[pl.ds(NBF * 64, TAIL_COLS // 2)], osem0).wait()


def _gather_kernel(packed, tokT, out, idx0, idx1, pb0, pb1, pr0, pr1,
                   ob0, ob1, is0, is1, gs0, gs1, os0, os1):
    w = lax.axis_index("s") * NC + lax.axis_index("c")
    idxs = (idx0, idx1)
    pbs = (pb0, pb1)
    prs = (pr0, pr1)
    obs = (ob0, ob1)
    isems = (is0, is1)
    gsems = (gs0, gs1)
    osems = (os0, os1)
    col = w * 128

    def idx_src(t):
        return tokT.at[t, pl.ds(col, 128)]

    def out_dst(t):
        return out.at[t, :, pl.ds(col, 128)]

    def prep_pair(idxb, pairb):
        for k in range(8):
            s16 = pl.ds(16 * k, 16)
            pairb[s16] = lax.shift_right_logical(idxb[s16], 1)

    def transpose_scale(idxb, pairs, outb):
        hvs = []
        r_idxs = []
        for k in range(8):
            s16 = pl.ds(16 * k, 16)
            hvs.append((idxb[s16] & 1) * 64)
            r_idxs.append(_iota16() + 16 * k)

        def fbody(f, c):
            for k in range(8):
                v = plsc.load_gather(pairs, [r_idxs[k], hvs[k] + f])
                outb[f, pl.ds(16 * k, 16)] = v * SCALE
            return c

        lax.fori_loop(0, EMB, fbody, 0, unroll=4)

    pltpu.async_copy(idx_src(0), idx0, is0)
    pltpu.async_copy(idx_src(1), idx1, is1)
    pltpu.make_async_copy(idx_src(0), idx0, is0).wait()
    def pr_dst(pairs):
        return pairs.at[:, pl.ds(0, 128)]

    prep_pair(idx0, pb0)
    pltpu.async_copy(packed.at[pb0], pr_dst(pr0), gs0)

    def group(g, carry):
        for b in range(2):
            t = g * 2 + b
            idxb, pairb, pairs, outb = idxs[b], pbs[b], prs[b], obs[b]
            isem, gsem, osem = isems[b], gsems[b], osems[b]

            @pl.when(t >= 2)
            def _():
                pltpu.make_async_copy(outb, out_dst(t - 2), osem).wait()

            pltpu.make_async_copy(
                packed.at[pairb], pr_dst(pairs), gsem).wait()
            transpose_scale(idxb, pairs, outb)
            pltpu.async_copy(outb, out_dst(t), osem)

            @pl.when(t + 1 <= T - 1)
            def _():
                pltpu.make_async_copy(
                    idx_src(t + 1), idxs[1 - b], isems[1 - b]).wait()
                prep_pair(idxs[1 - b], pbs[1 - b])
                pltpu.async_copy(packed.at[pbs[1 - b]], pr_dst(prs[1 - b]),
                                 gsems[1 - b])

            @pl.when(t + 2 <= T - 1)
            def _():
                pltpu.async_copy(idx_src(t + 2), idxb, isem)
        return carry

    lax.fori_loop(0, T // 2, group, 0)

    pltpu.make_async_copy(ob0, out_dst(T - 2), os0).wait()
    pltpu.make_async_copy(ob1, out_dst(T - 1), os1).wait()


@jax.jit
def _emb_lookup(tokT, tabT, tail2):
    mesh = plsc.VectorSubcoreMesh(core_axis_name="c", subcore_axis_name="s")
    cp = pltpu.CompilerParams(needs_layout_passes=False)
    pack = functools.partial(
        pl.kernel,
        out_type=jax.ShapeDtypeStruct((NPB, 128), jnp.float32),
        mesh=mesh,
        scratch_types=[
            pltpu.VMEM((EMB, 129), jnp.float32),
            pltpu.VMEM((EMB, 129), jnp.float32),
            pltpu.VMEM((EMB, 128), jnp.float32),
            pltpu.VMEM((EMB, 128), jnp.float32),
            pltpu.SemaphoreType.DMA,
            pltpu.SemaphoreType.DMA,
            pltpu.SemaphoreType.DMA,
            pltpu.SemaphoreType.DMA,
        ],
        compiler_params=cp,
    )(_pack_kernel)
    packed = pack(tabT, tail2)

    gather = functools.partial(
        pl.kernel,
        out_type=jax.ShapeDtypeStruct((T, EMB, S), jnp.float32),
        mesh=mesh,
        scratch_types=[
            pltpu.VMEM((128,), jnp.int32),
            pltpu.VMEM((128,), jnp.int32),
            pltpu.VMEM((128,), jnp.int32),
            pltpu.VMEM((128,), jnp.int32),
            pltpu.VMEM((128, 129), jnp.float32),
            pltpu.VMEM((128, 129), jnp.float32),
            pltpu.VMEM((EMB, 128), jnp.float32),
            pltpu.VMEM((EMB, 128), jnp.float32),
            pltpu.SemaphoreType.DMA,
            pltpu.SemaphoreType.DMA,
            pltpu.SemaphoreType.DMA,
            pltpu.SemaphoreType.DMA,
            pltpu.SemaphoreType.DMA,
            pltpu.SemaphoreType.DMA,
        ],
        compiler_params=cp,
    )(_gather_kernel)
    return gather(packed, tokT)


def kernel(tokens, table):
    tail2 = table[NBF * 128:].reshape(TAIL_COLS // 2, 128)
    out3 = _emb_lookup(tokens.T.astype(jnp.int32), table.T, tail2)
    return out3.transpose(2, 0, 1)


# speedup vs baseline: 1.0590x; 1.0590x over previous
import functools
import math

import jax
import jax.numpy as jnp
from jax import lax
from jax.experimental import pallas as pl
from jax.experimental.pallas import tpu as pltpu
from jax.experimental.pallas import tpu_sc as plsc

VOCAB = 1000000
EMB = 64
SCALE = math.sqrt(EMB)

NC = 2
NS = 16
NW = NC * NS

S = 4096
T = 200

NPB = VOCAB // 256 * 128 + 64
NBF = VOCAB // 128
TAIL_COLS = VOCAB - NBF * 128
NG1 = 123


def _iota16():
    return lax.iota(jnp.int32, 16)


def _pack_kernel(tabT, tail2, packed, inb0, inb1, outb0, outb1,
                 isem0, isem1, osem0, osem1):
    w = lax.axis_index("s") * NC + lax.axis_index("c")
    inbs = (inb0, inb1)
    outbs = (outb0, outb1)
    isems = (isem0, isem1)
    osems = (osem0, osem1)

    def blk(k):
        return w + 32 * k

    def valid(k):
        return blk(k) <= NBF - 1

    r_idx4 = [_iota16() + 16 * k for k in range(4)]

    def transpose_pack(inb, outb, nrows):
        def ibody(i, c):
            c_even = jnp.broadcast_to(2 * i, (16,)).astype(jnp.int32)
            c_odd = c_even + 1
            for k in range(8):
                r_idx = r_idx4[k % 4]
                c_idx = c_even if k < 4 else c_odd
                v = plsc.load_gather(inb, [r_idx, c_idx])
                outb[i, pl.ds(16 * k, 16)] = v
            return c
        lax.fori_loop(0, nrows, ibody, 0, unroll=4)

    def inb_dst(inb):
        return inb.at[:, pl.ds(0, 128)]

    @pl.when(valid(0))
    def _():
        pltpu.async_copy(
            tabT.at[:, pl.ds(blk(0) * 128, 128)], inb_dst(inb0), isem0)

    def group(g, carry):
        for b in range(2):
            k = g * 2 + b
            inb, outb, isem, osem = inbs[b], outbs[b], isems[b], osems[b]

            @pl.when(valid(k))
            def _():
                v0 = blk(k)
                pltpu.make_async_copy(
                    tabT.at[:, pl.ds(v0 * 128, 128)], inb_dst(inb),
                    isem).wait()

                @pl.when(valid(k + 1))
                def _():
                    pltpu.async_copy(
                        tabT.at[:, pl.ds(blk(k + 1) * 128, 128)],
                        inb_dst(inbs[1 - b]), isems[1 - b])

                @pl.when(k >= 2)
                def _():
                    pltpu.make_async_copy(
                        outb, packed.at[pl.ds(blk(k - 2) * 64, 64)], osem
                    ).wait()

                transpose_pack(inb, outb, 64)
                pltpu.async_copy(outb, packed.at[pl.ds(v0 * 64, 64)], osem)
        return carry

    lax.fori_loop(0, NG1, group, 0)

    @pl.when(w < 4)
    def _():
        pltpu.make_async_copy(
            outb1, packed.at[pl.ds(blk(243) * 64, 64)], osem1).wait()
        pltpu.make_async_copy(
            outb0, packed.at[pl.ds(blk(244) * 64, 64)], osem0).wait()

    @pl.when(w >= 4)
    def _():
        pltpu.make_async_copy(
            outb0, packed.at[pl.ds(blk(242) * 64, 64)], osem0).wait()
        pltpu.make_async_copy(
            outb1, packed.at[pl.ds(blk(243) * 64, 64)], osem1).wait()

    @pl.when(w == 4)
    def _():
        pltpu.async_copy(
            tail2, inb0.at[pl.ds(0, TAIL_COLS // 2), pl.ds(0, 128)],
            isem0).wait()
        pltpu.async_copy(
            inb0.at[pl.ds(0, TAIL_COLS // 2), pl.ds(0, 128)],
            packed.at[pl.ds(NBF * 64, TAIL_COLS // 2)], osem0).wait()


def _gather_kernel(packed, tokT, out, idx0, idx1, pb0, pb1, pr0, pr1,
                   ob0, ob1, is0, is1, gs0, gs1, os0, os1):
    w = lax.axis_index("s") * NC + lax.axis_index("c")
    idxs = (idx0, idx1)
    pbs = (pb0, pb1)
    prs = (pr0, pr1)
    obs = (ob0, ob1)
    isems = (is0, is1)
    gsems = (gs0, gs1)
    osems = (os0, os1)
    col = w * 128

    def idx_src(t):
        return tokT.at[t, pl.ds(col, 128)]

    def out_dst(t):
        return out.at[t, :, pl.ds(col, 128)]

    def prep_pair(idxb, pairb):
        for k in range(8):
            s16 = pl.ds(16 * k, 16)
            pairb[s16] = lax.shift_right_logical(idxb[s16], 1)

    def transpose_scale(idxb, pairs, outb):
        hvs = []
        r_idxs = []
        for k in range(8):
            s16 = pl.ds(16 * k, 16)
            hvs.append((idxb[s16] & 1) * 64)
            r_idxs.append(_iota16() + 16 * k)

        def fbody(f, c):
            for k in range(8):
                v = plsc.load_gather(pairs, [r_idxs[k], hvs[k] + f])
                outb[f, pl.ds(16 * k, 16)] = v * SCALE
            return c

        lax.fori_loop(0, EMB, fbody, 0, unroll=4)

    pltpu.async_copy(idx_src(0), idx0, is0)
    pltpu.async_copy(idx_src(1), idx1, is1)
    pltpu.make_async_copy(idx_src(0), idx0, is0).wait()
    def pr_dst(pairs):
        return pairs.at[:, pl.ds(0, 128)]

    prep_pair(idx0, pb0)
    pltpu.async_copy(packed.at[pb0], pr_dst(pr0), gs0)

    def group(g, carry):
        for b in range(2):
            t = g * 2 + b
            idxb, pairb, pairs, outb = idxs[b], pbs[b], prs[b], obs[b]
            isem, gsem, osem = isems[b], gsems[b], osems[b]

            @pl.when(t + 1 <= T - 1)
            def _():
                pltpu.make_async_copy(
                    idx_src(t + 1), idxs[1 - b], isems[1 - b]).wait()
                prep_pair(idxs[1 - b], pbs[1 - b])
                pltpu.async_copy(packed.at[pbs[1 - b]], pr_dst(prs[1 - b]),
                                 gsems[1 - b])

            @pl.when(t >= 2)
            def _():
                pltpu.make_async_copy(outb, out_dst(t - 2), osem).wait()

            pltpu.make_async_copy(
                packed.at[pairb], pr_dst(pairs), gsem).wait()
            transpose_scale(idxb, pairs, outb)
            pltpu.async_copy(outb, out_dst(t), osem)

            @pl.when(t + 2 <= T - 1)
            def _():
                pltpu.async_copy(idx_src(t + 2), idxb, isem)
        return carry

    lax.fori_loop(0, T // 2, group, 0)

    pltpu.make_async_copy(ob0, out_dst(T - 2), os0).wait()
    pltpu.make_async_copy(ob1, out_dst(T - 1), os1).wait()


@jax.jit
def _emb_lookup(tokT, tabT, tail2):
    mesh = plsc.VectorSubcoreMesh(core_axis_name="c", subcore_axis_name="s")
    cp = pltpu.CompilerParams(needs_layout_passes=False)
    pack = functools.partial(
        pl.kernel,
        out_type=jax.ShapeDtypeStruct((NPB, 128), jnp.float32),
        mesh=mesh,
        scratch_types=[
            pltpu.VMEM((EMB, 129), jnp.float32),
            pltpu.VMEM((EMB, 129), jnp.float32),
            pltpu.VMEM((EMB, 128), jnp.float32),
            pltpu.VMEM((EMB, 128), jnp.float32),
            pltpu.SemaphoreType.DMA,
            pltpu.SemaphoreType.DMA,
            pltpu.SemaphoreType.DMA,
            pltpu.SemaphoreType.DMA,
        ],
        compiler_params=cp,
    )(_pack_kernel)
    packed = pack(tabT, tail2)

    gather = functools.partial(
        pl.kernel,
        out_type=jax.ShapeDtypeStruct((T, EMB, S), jnp.float32),
        mesh=mesh,
        scratch_types=[
            pltpu.VMEM((128,), jnp.int32),
            pltpu.VMEM((128,), jnp.int32),
            pltpu.VMEM((128,), jnp.int32),
            pltpu.VMEM((128,), jnp.int32),
            pltpu.VMEM((128, 129), jnp.float32),
            pltpu.VMEM((128, 129), jnp.float32),
            pltpu.VMEM((EMB, 128), jnp.float32),
            pltpu.VMEM((EMB, 128), jnp.float32),
            pltpu.SemaphoreType.DMA,
            pltpu.SemaphoreType.DMA,
            pltpu.SemaphoreType.DMA,
            pltpu.SemaphoreType.DMA,
            pltpu.SemaphoreType.DMA,
            pltpu.SemaphoreType.DMA,
        ],
        compiler_params=cp,
    )(_gather_kernel)
    return gather(packed, tokT)


def kernel(tokens, table):
    tail2 = table[NBF * 128:].reshape(TAIL_COLS // 2, 128)
    out3 = _emb_lookup(tokens.T.astype(jnp.int32), table.T, tail2)
    return out3.transpose(2, 0, 1)


# speedup vs baseline: 2.0031x; 1.8914x over previous
import functools
import math

import jax
import jax.numpy as jnp
from jax import lax
from jax.experimental import pallas as pl
from jax.experimental.pallas import tpu as pltpu
from jax.experimental.pallas import tpu_sc as plsc

VOCAB = 1000000
EMB = 64
SCALE = math.sqrt(EMB)

NC = 2
NS = 16
NW = NC * NS

B = 4096 * 200
BPW = B // NW
CH = 128
NCHUNK = BPW // CH


def _emb_kernel(table_hbm, idx_hbm, out_hbm, idx_v,
                gb0, gb1, ob0, ob1, gs0, gs1, os0, os1):
    wid = lax.axis_index("s") * NC + lax.axis_index("c")
    base = wid * BPW
    gbs = (gb0, gb1)
    obs = (ob0, ob1)
    gss = (gs0, gs1)
    oss = (os0, os1)

    pltpu.sync_copy(idx_hbm.at[pl.ds(wid * NCHUNK, NCHUNK)], idx_v)

    pltpu.async_copy(table_hbm.at[idx_v.at[0]], gb0, gs0)

    def group(g, carry):
        for b in range(2):
            j = g * 2 + b
            gb, ob, gsem, osem = gbs[b], obs[b], gss[b], oss[b]

            @pl.when(j + 1 < NCHUNK)
            def _():
                pltpu.async_copy(
                    table_hbm.at[idx_v.at[j + 1]], gbs[1 - b], gss[1 - b])

            pltpu.make_async_copy(table_hbm.at[idx_v.at[j]], gb, gsem).wait()

            @pl.when(j >= 2)
            def _():
                pltpu.make_async_copy(
                    ob, out_hbm.at[pl.ds(base + (j - 2) * CH, CH)], osem
                ).wait()

            def scale_body(i, c):
                for q in range(EMB // 16):
                    s = pl.ds(q * 16, 16)
                    ob[i, s] = gb[i, s] * SCALE
                return c

            lax.fori_loop(0, CH, scale_body, 0, unroll=8)
            pltpu.async_copy(ob, out_hbm.at[pl.ds(base + j * CH, CH)], osem)
        return carry

    lax.fori_loop(0, NCHUNK // 2, group, 0)

    for b in range(2):
        j = NCHUNK - 2 + b
        pltpu.make_async_copy(
            obs[b], out_hbm.at[pl.ds(base + j * CH, CH)], oss[b]).wait()


@jax.jit
def _emb_lookup(idx2d, table):
    mesh = plsc.VectorSubcoreMesh(core_axis_name="c", subcore_axis_name="s")
    fn = functools.partial(
        pl.kernel,
        out_type=jax.ShapeDtypeStruct((B, EMB), jnp.float32),
        mesh=mesh,
        scratch_types=[
            pltpu.VMEM((NCHUNK, CH), jnp.int32),
            pltpu.VMEM((CH, EMB), jnp.float32),
            pltpu.VMEM((CH, EMB), jnp.float32),
            pltpu.VMEM((CH, EMB), jnp.float32),
            pltpu.VMEM((CH, EMB), jnp.float32),
            pltpu.SemaphoreType.DMA,
            pltpu.SemaphoreType.DMA,
            pltpu.SemaphoreType.DMA,
            pltpu.SemaphoreType.DMA,
        ],
        compiler_params=pltpu.CompilerParams(use_tc_tiling_on_sc=False),
    )(_emb_kernel)
    return fn(table, idx2d)


def kernel(tokens, table):
    idx2d = tokens.reshape(-1).astype(jnp.int32).reshape(NW * NCHUNK, CH)
    out = _emb_lookup(idx2d, table)
    return out.reshape(tokens.shape[0], tokens.shape[1], EMB)


# speedup vs baseline: 2.4415x; 1.2189x over previous
import functools
import math

import jax
import jax.numpy as jnp
from jax import lax
from jax.experimental import pallas as pl
from jax.experimental.pallas import tpu as pltpu
from jax.experimental.pallas import tpu_sc as plsc

VOCAB = 1000000
EMB = 64
SCALE = math.sqrt(EMB)

NC = 2
NS = 16
NW = NC * NS

B = 4096 * 200
BPW = B // NW
CH = 128
NCHUNK = BPW // CH


def _emb_kernel(table_hbm, idx_hbm, out_hbm, idx_v, gb0, gb1, gs0, gs1):
    wid = lax.axis_index("s") * NC + lax.axis_index("c")
    base = wid * BPW
    gbs = (gb0, gb1)
    gss = (gs0, gs1)

    pltpu.sync_copy(idx_hbm.at[pl.ds(wid * NCHUNK, NCHUNK)], idx_v)

    pltpu.async_copy(table_hbm.at[idx_v.at[0]], gb0, gs0)

    def group(g, carry):
        for b in range(2):
            j = g * 2 + b
            gb, gsem = gbs[b], gss[b]

            @pl.when(j + 1 < NCHUNK)
            def _():
                pltpu.async_copy(
                    table_hbm.at[idx_v.at[j + 1]], gbs[1 - b], gss[1 - b])

            pltpu.make_async_copy(table_hbm.at[idx_v.at[j]], gb, gsem).wait()

            def scale_body(i, c):
                for q in range(EMB // 16):
                    s = pl.ds(q * 16, 16)
                    gb[i, s] = gb[i, s] * SCALE
                return c

            lax.fori_loop(0, CH, scale_body, 0, unroll=8)
            pltpu.sync_copy(gb, out_hbm.at[pl.ds(base + j * CH, CH)])
        return carry

    lax.fori_loop(0, NCHUNK // 2, group, 0)


@jax.jit
def _emb_lookup(idx2d, table):
    mesh = plsc.VectorSubcoreMesh(core_axis_name="c", subcore_axis_name="s")
    fn = functools.partial(
        pl.kernel,
        out_type=jax.ShapeDtypeStruct((B, EMB), jnp.float32),
        mesh=mesh,
        scratch_types=[
            pltpu.VMEM((NCHUNK, CH), jnp.int32),
            pltpu.VMEM((CH, EMB), jnp.float32),
            pltpu.VMEM((CH, EMB), jnp.float32),
            pltpu.SemaphoreType.DMA,
            pltpu.SemaphoreType.DMA,
        ],
        compiler_params=pltpu.CompilerParams(use_tc_tiling_on_sc=False),
    )(_emb_kernel)
    return fn(table, idx2d)


def kernel(tokens, table):
    idx2d = tokens.reshape(-1).astype(jnp.int32).reshape(NW * NCHUNK, CH)
    out = _emb_lookup(idx2d, table)
    return out.reshape(tokens.shape[0], tokens.shape[1], EMB)


# speedup vs baseline: 2.5167x; 1.0308x over previous
import functools
import math

import jax
import jax.numpy as jnp
from jax import lax
from jax.experimental import pallas as pl
from jax.experimental.pallas import tpu as pltpu
from jax.experimental.pallas import tpu_sc as plsc

VOCAB = 1000000
EMB = 64
SCALE = math.sqrt(EMB)

NC = 2
NS = 16
NW = NC * NS

B = 4096 * 200
BPW = B // NW
CH = 128
NCHUNK = BPW // CH


def _emb_kernel(table_hbm, idx_hbm, out_hbm, idx_v,
                gb0, gb1, gb2, gs0, gs1, gs2):
    wid = lax.axis_index("s") * NC + lax.axis_index("c")
    base = wid * BPW
    gbs = (gb0, gb1, gb2)
    gss = (gs0, gs1, gs2)

    pltpu.sync_copy(idx_hbm.at[pl.ds(wid * NCHUNK, NCHUNK)], idx_v)

    pltpu.async_copy(table_hbm.at[idx_v.at[0]], gb0, gs0)
    pltpu.async_copy(table_hbm.at[idx_v.at[1]], gb1, gs1)

    def group(g, carry):
        for b in range(3):
            j = g * 3 + b

            @pl.when(j < NCHUNK)
            def _():
                gb, gsem = gbs[b], gss[b]

                @pl.when(j + 2 < NCHUNK)
                def _():
                    nb = (b + 2) % 3
                    pltpu.async_copy(
                        table_hbm.at[idx_v.at[j + 2]], gbs[nb], gss[nb])

                pltpu.make_async_copy(
                    table_hbm.at[idx_v.at[j]], gb, gsem).wait()

                def scale_body(i, c):
                    for q in range(EMB // 16):
                        s = pl.ds(q * 16, 16)
                        gb[i, s] = gb[i, s] * SCALE
                    return c

                lax.fori_loop(0, CH, scale_body, 0, unroll=8)
                pltpu.sync_copy(gb, out_hbm.at[pl.ds(base + j * CH, CH)])
        return carry

    lax.fori_loop(0, (NCHUNK + 2) // 3, group, 0)


@jax.jit
def _emb_lookup(idx2d, table):
    mesh = plsc.VectorSubcoreMesh(core_axis_name="c", subcore_axis_name="s")
    fn = functools.partial(
        pl.kernel,
        out_type=jax.ShapeDtypeStruct((B, EMB), jnp.float32),
        mesh=mesh,
        scratch_types=[
            pltpu.VMEM((NCHUNK, CH), jnp.int32),
            pltpu.VMEM((CH, EMB), jnp.float32),
            pltpu.VMEM((CH, EMB), jnp.float32),
            pltpu.VMEM((CH, EMB), jnp.float32),
            pltpu.SemaphoreType.DMA,
            pltpu.SemaphoreType.DMA,
            pltpu.SemaphoreType.DMA,
        ],
        compiler_params=pltpu.CompilerParams(use_tc_tiling_on_sc=False),
    )(_emb_kernel)
    return fn(table, idx2d)


def kernel(tokens, table):
    idx2d = tokens.reshape(-1).astype(jnp.int32).reshape(NW * NCHUNK, CH)
    out = _emb_lookup(idx2d, table)
    return out.reshape(tokens.shape[0], tokens.shape[1], EMB)
